# baseline (speedup 1.0000x reference)
import jax
import jax.numpy as jnp
from jax.experimental import pallas as pl

BB = 8


def _body(edges_ref, atoms_ref, w1_ref, w2_ref, aux_ref, sel_ref, iota_ref,
          summed_ref, out1_ref, out2_ref):
    bb, n, k = edges_ref.shape
    d = atoms_ref.shape[-1]
    kcw = w1_ref.shape[-1]
    cw = kcw // k
    iota_f = iota_ref[...]
    ones_1n = jnp.ones((1, n), jnp.float32)
    b1 = aux_ref[0:1, :]
    b2 = aux_ref[1:2, :]
    jdiv = aux_ref[2:3, :]
    for i in range(bb):
        e_f = edges_ref[i]
        a_cnt = jnp.zeros((n, n), jnp.float32)
        for j in range(k):
            col = jax.lax.slice(e_f, (0, j), (n, j + 1))
            col_b = jnp.dot(col, ones_1n,
                            preferred_element_type=jnp.float32)
            a_cnt += (col_b == iota_f).astype(jnp.float32)
        summed_ref[i] = jnp.dot(a_cnt, atoms_ref[i],
                                preferred_element_type=jnp.float32)
    e_all = edges_ref[...].reshape(bb * n, k)
    mask_e = (e_all != -1.0).astype(jnp.float32)
    ones_kj = jnp.ones((k, kcw), jnp.float32)
    deg_b = jnp.dot(mask_e, ones_kj,
                    preferred_element_type=jnp.float32)
    m_sel = (deg_b == jdiv).astype(jnp.float32)
    s_all = summed_ref[...].reshape(bb * n, d)
    a_all = atoms_ref[...].reshape(bb * n, d)
    z1 = jnp.maximum(jnp.dot(s_all, w1_ref[...],
                             preferred_element_type=jnp.float32) + b1, 0.0)
    z2 = jnp.maximum(jnp.dot(a_all, w2_ref[...],
                             preferred_element_type=jnp.float32) + b2, 0.0)
    o1 = jnp.dot(z1 * m_sel, sel_ref[...],
                 preferred_element_type=jnp.float32)
    o2 = jnp.dot(z2 * m_sel, sel_ref[...],
                 preferred_element_type=jnp.float32)
    out1_ref[...] = o1.reshape(bb, n, cw)
    out2_ref[...] = o2.reshape(bb, n, cw)


def kernel(atoms, edges, W1, b1, W2, b2):
    B, N, D = atoms.shape
    K = edges.shape[-1]
    CW = W1.shape[-1]
    w1r = W1.transpose(1, 0, 2).reshape(D, K * CW)
    w2r = W2.transpose(1, 0, 2).reshape(D, K * CW)
    aux = jnp.zeros((8, K * CW), jnp.float32)
    aux = aux.at[0].set(b1.reshape(-1)).at[1].set(b2.reshape(-1))
    aux = aux.at[2].set(jnp.repeat(jnp.arange(K, dtype=jnp.float32), CW))
    sel = jnp.tile(jnp.eye(CW, dtype=jnp.float32), (K, 1))
    edges_f = edges.astype(jnp.float32)
    iota_f = jnp.broadcast_to(jnp.arange(N, dtype=jnp.float32)[None, :], (N, N))
    summed, o1, o2 = pl.pallas_call(
        _body,
        grid=(B // BB,),
        in_specs=[
            pl.BlockSpec((BB, N, K), lambda i: (i, 0, 0)),
            pl.BlockSpec((BB, N, D), lambda i: (i, 0, 0)),
            pl.BlockSpec((D, K * CW), lambda i: (0, 0)),
            pl.BlockSpec((D, K * CW), lambda i: (0, 0)),
            pl.BlockSpec((8, K * CW), lambda i: (0, 0)),
            pl.BlockSpec((K * CW, CW), lambda i: (0, 0)),
            pl.BlockSpec((N, N), lambda i: (0, 0)),
        ],
        out_specs=[
            pl.BlockSpec((BB, N, D), lambda i: (i, 0, 0)),
            pl.BlockSpec((BB, N, CW), lambda i: (i, 0, 0)),
            pl.BlockSpec((BB, N, CW), lambda i: (i, 0, 0)),
        ],
        out_shape=[
            jax.ShapeDtypeStruct((B, N, D), jnp.float32),
            jax.ShapeDtypeStruct((B, N, CW), jnp.float32),
            jax.ShapeDtypeStruct((B, N, CW), jnp.float32),
        ],
    )(edges_f, atoms, w1r, w2r, aux, sel, iota_f)
    return (summed, atoms, o1, o2)


# speedup vs baseline: 1.0475x; 1.0475x over previous
import jax
import jax.numpy as jnp
from jax.experimental import pallas as pl

BB = 16


def _body(edges_ref, atoms_ref, w1_ref, w2_ref, aux_ref, sel_ref, iota_ref,
          summed_ref, out1_ref, out2_ref):
    bb, n, k = edges_ref.shape
    d = atoms_ref.shape[-1]
    kcw = w1_ref.shape[-1]
    cw = kcw // k
    iota_f = iota_ref[...]
    ones_1n = jnp.ones((1, n), jnp.float32)
    b1 = aux_ref[0:1, :]
    b2 = aux_ref[1:2, :]
    jdiv = aux_ref[2:3, :]
    for i in range(bb):
        e_f = edges_ref[i]
        a_cnt = jnp.zeros((n, n), jnp.float32)
        for j in range(k):
            col = jax.lax.slice(e_f, (0, j), (n, j + 1))
            col_b = jnp.dot(col, ones_1n,
                            preferred_element_type=jnp.float32)
            a_cnt += (col_b == iota_f).astype(jnp.float32)
        summed_ref[i] = jnp.dot(a_cnt, atoms_ref[i],
                                preferred_element_type=jnp.float32)
    e_all = edges_ref[...].reshape(bb * n, k)
    mask_e = (e_all != -1.0).astype(jnp.float32)
    ones_kj = jnp.ones((k, kcw), jnp.float32)
    deg_b = jnp.dot(mask_e, ones_kj,
                    preferred_element_type=jnp.float32)
    m_sel = (deg_b == jdiv).astype(jnp.float32)
    s_all = summed_ref[...].reshape(bb * n, d)
    a_all = atoms_ref[...].reshape(bb * n, d)
    z1 = jnp.maximum(jnp.dot(s_all, w1_ref[...],
                             preferred_element_type=jnp.float32) + b1, 0.0)
    z2 = jnp.maximum(jnp.dot(a_all, w2_ref[...],
                             preferred_element_type=jnp.float32) + b2, 0.0)
    o1 = jnp.dot(z1 * m_sel, sel_ref[...],
                 preferred_element_type=jnp.float32)
    o2 = jnp.dot(z2 * m_sel, sel_ref[...],
                 preferred_element_type=jnp.float32)
    out1_ref[...] = o1.reshape(bb, n, cw)
    out2_ref[...] = o2.reshape(bb, n, cw)


def kernel(atoms, edges, W1, b1, W2, b2):
    B, N, D = atoms.shape
    K = edges.shape[-1]
    CW = W1.shape[-1]
    w1r = W1.transpose(1, 0, 2).reshape(D, K * CW)
    w2r = W2.transpose(1, 0, 2).reshape(D, K * CW)
    aux = jnp.zeros((8, K * CW), jnp.float32)
    aux = aux.at[0].set(b1.reshape(-1)).at[1].set(b2.reshape(-1))
    aux = aux.at[2].set(jnp.repeat(jnp.arange(K, dtype=jnp.float32), CW))
    sel = jnp.tile(jnp.eye(CW, dtype=jnp.float32), (K, 1))
    edges_f = edges.astype(jnp.float32)
    iota_f = jnp.broadcast_to(jnp.arange(N, dtype=jnp.float32)[None, :], (N, N))
    summed, o1, o2 = pl.pallas_call(
        _body,
        grid=(B // BB,),
        in_specs=[
            pl.BlockSpec((BB, N, K), lambda i: (i, 0, 0)),
            pl.BlockSpec((BB, N, D), lambda i: (i, 0, 0)),
            pl.BlockSpec((D, K * CW), lambda i: (0, 0)),
            pl.BlockSpec((D, K * CW), lambda i: (0, 0)),
            pl.BlockSpec((8, K * CW), lambda i: (0, 0)),
            pl.BlockSpec((K * CW, CW), lambda i: (0, 0)),
            pl.BlockSpec((N, N), lambda i: (0, 0)),
        ],
        out_specs=[
            pl.BlockSpec((BB, N, D), lambda i: (i, 0, 0)),
            pl.BlockSpec((BB, N, CW), lambda i: (i, 0, 0)),
            pl.BlockSpec((BB, N, CW), lambda i: (i, 0, 0)),
        ],
        out_shape=[
            jax.ShapeDtypeStruct((B, N, D), jnp.float32),
            jax.ShapeDtypeStruct((B, N, CW), jnp.float32),
            jax.ShapeDtypeStruct((B, N, CW), jnp.float32),
        ],
    )(edges_f, atoms, w1r, w2r, aux, sel, iota_f)
    return (summed, atoms, o1, o2)


# speedup vs baseline: 1.0620x; 1.0139x over previous
import jax
import jax.numpy as jnp
from jax.experimental import pallas as pl

BB = 32


def _body(edges_ref, atoms_ref, w1_ref, w2_ref, aux_ref, sel_ref, iota_ref,
          summed_ref, out1_ref, out2_ref):
    bb, n, k = edges_ref.shape
    d = atoms_ref.shape[-1]
    kcw = w1_ref.shape[-1]
    cw = kcw // k
    iota_f = iota_ref[...]
    ones_1n = jnp.ones((1, n), jnp.float32)
    b1 = aux_ref[0:1, :]
    b2 = aux_ref[1:2, :]
    jdiv = aux_ref[2:3, :]
    for i in range(bb):
        e_f = edges_ref[i]
        a_cnt = jnp.zeros((n, n), jnp.float32)
        for j in range(k):
            col = jax.lax.slice(e_f, (0, j), (n, j + 1))
            col_b = jnp.dot(col, ones_1n,
                            preferred_element_type=jnp.float32)
            a_cnt += (col_b == iota_f).astype(jnp.float32)
        summed_ref[i] = jnp.dot(a_cnt, atoms_ref[i],
                                preferred_element_type=jnp.float32)
    e_all = edges_ref[...].reshape(bb * n, k)
    mask_e = (e_all != -1.0).astype(jnp.float32)
    ones_kj = jnp.ones((k, kcw), jnp.float32)
    deg_b = jnp.dot(mask_e, ones_kj,
                    preferred_element_type=jnp.float32)
    m_sel = (deg_b == jdiv).astype(jnp.float32)
    s_all = summed_ref[...].reshape(bb * n, d)
    a_all = atoms_ref[...].reshape(bb * n, d)
    z1 = jnp.maximum(jnp.dot(s_all, w1_ref[...],
                             preferred_element_type=jnp.float32) + b1, 0.0)
    z2 = jnp.maximum(jnp.dot(a_all, w2_ref[...],
                             preferred_element_type=jnp.float32) + b2, 0.0)
    o1 = jnp.dot(z1 * m_sel, sel_ref[...],
                 preferred_element_type=jnp.float32)
    o2 = jnp.dot(z2 * m_sel, sel_ref[...],
                 preferred_element_type=jnp.float32)
    out1_ref[...] = o1.reshape(bb, n, cw)
    out2_ref[...] = o2.reshape(bb, n, cw)


def kernel(atoms, edges, W1, b1, W2, b2):
    B, N, D = atoms.shape
    K = edges.shape[-1]
    CW = W1.shape[-1]
    w1r = W1.transpose(1, 0, 2).reshape(D, K * CW)
    w2r = W2.transpose(1, 0, 2).reshape(D, K * CW)
    aux = jnp.zeros((8, K * CW), jnp.float32)
    aux = aux.at[0].set(b1.reshape(-1)).at[1].set(b2.reshape(-1))
    aux = aux.at[2].set(jnp.repeat(jnp.arange(K, dtype=jnp.float32), CW))
    sel = jnp.tile(jnp.eye(CW, dtype=jnp.float32), (K, 1))
    edges_f = edges.astype(jnp.float32)
    iota_f = jnp.broadcast_to(jnp.arange(N, dtype=jnp.float32)[None, :], (N, N))
    summed, o1, o2 = pl.pallas_call(
        _body,
        grid=(B // BB,),
        in_specs=[
            pl.BlockSpec((BB, N, K), lambda i: (i, 0, 0)),
            pl.BlockSpec((BB, N, D), lambda i: (i, 0, 0)),
            pl.BlockSpec((D, K * CW), lambda i: (0, 0)),
            pl.BlockSpec((D, K * CW), lambda i: (0, 0)),
            pl.BlockSpec((8, K * CW), lambda i: (0, 0)),
            pl.BlockSpec((K * CW, CW), lambda i: (0, 0)),
            pl.BlockSpec((N, N), lambda i: (0, 0)),
        ],
        out_specs=[
            pl.BlockSpec((BB, N, D), lambda i: (i, 0, 0)),
            pl.BlockSpec((BB, N, CW), lambda i: (i, 0, 0)),
            pl.BlockSpec((BB, N, CW), lambda i: (i, 0, 0)),
        ],
        out_shape=[
            jax.ShapeDtypeStruct((B, N, D), jnp.float32),
            jax.ShapeDtypeStruct((B, N, CW), jnp.float32),
            jax.ShapeDtypeStruct((B, N, CW), jnp.float32),
        ],
    )(edges_f, atoms, w1r, w2r, aux, sel, iota_f)
    return (summed, atoms, o1, o2)
